# baseline (speedup 1.0000x reference)
import functools

import jax
import jax.numpy as jnp
from jax import lax
from jax.experimental import pallas as pl
from jax.experimental.pallas import tpu as pltpu
from jax.experimental.pallas import tpu_sc as plsc

BATCH = 4096
HIST = 200
D = 64

NC, NS = 2, 16
NW = NC * NS
BW = BATCH // NW
SROWS = 500000

_mesh = plsc.VectorSubcoreMesh(core_axis_name="c", subcore_axis_name="s")


@functools.partial(
    pl.kernel,
    out_type=jax.ShapeDtypeStruct((HIST, D, BATCH), jnp.float32),
    mesh=_mesh,
    scratch_types=[
        pltpu.VMEM((HIST, 1, BW), jnp.int32),
        pltpu.VMEM((BW,), jnp.int32),
        pltpu.VMEM((BW,), jnp.int32),
        pltpu.VMEM((BW, 128), jnp.float32),
        pltpu.VMEM((D, BW), jnp.float32),
        pltpu.SemaphoreType.DMA,
    ],
    compiler_params=pltpu.CompilerParams(needs_layout_passes=False),
)
def _emb_kernel(idx_hbm, table_hbm, out_hbm, xblk, sidx, lbase, srows,
                oblk, sem):
    wid = lax.axis_index("s") * NC + lax.axis_index("c")

    pltpu.sync_copy(idx_hbm.at[:, pl.ds(wid, 1), :], xblk)

    iota = lax.iota(jnp.int32, 16)

    @pl.loop(0, HIST)
    def _h(h):
        row = xblk.at[h, 0]
        for c in range(BW // 16):
            v = row[pl.ds(c * 16, 16)]
            sidx[pl.ds(c * 16, 16)] = lax.shift_right_logical(v, 1)
            lbase[pl.ds(c * 16, 16)] = (v & 1) * 64
        pltpu.async_copy(table_hbm.at[sidx], srows, sem).wait()
        for c in range(BW // 16):
            rowv = c * 16 + iota
            colb = lbase[pl.ds(c * 16, 16)]

            @pl.loop(0, D)
            def _d(d):
                vals = plsc.load_gather(srows, [rowv, colb + d])
                oblk[d, pl.ds(c * 16, 16)] = vals

        pltpu.sync_copy(
            oblk, out_hbm.at[h, :, pl.ds(wid * BW, BW)])


def kernel(x, table):
    idx = x.T.reshape(HIST, NW, BW)
    table2 = table.reshape(SROWS, 2 * D)
    out = _emb_kernel(idx, table2)
    return out.transpose(2, 0, 1)


# speedup vs baseline: 1.0924x; 1.0924x over previous
import functools

import jax
import jax.numpy as jnp
from jax import lax
from jax.experimental import pallas as pl
from jax.experimental.pallas import tpu as pltpu
from jax.experimental.pallas import tpu_sc as plsc

BATCH = 4096
HIST = 200
D = 64
PD = 128

NC, NS = 2, 16
NW = NC * NS
BW = BATCH // NW
VOCAB = 1000000

_mesh = plsc.VectorSubcoreMesh(core_axis_name="c", subcore_axis_name="s")


@functools.partial(
    pl.kernel,
    out_type=jax.ShapeDtypeStruct((HIST, D, BATCH), jnp.float32),
    mesh=_mesh,
    scratch_types=[
        pltpu.VMEM((HIST, 1, BW), jnp.int32),
        pltpu.VMEM((BW, PD), jnp.float32),
        pltpu.VMEM((BW, PD), jnp.float32),
        pltpu.VMEM((D, BW), jnp.float32),
        pltpu.SemaphoreType.DMA,
        pltpu.SemaphoreType.DMA,
    ],
    compiler_params=pltpu.CompilerParams(needs_layout_passes=False),
)
def _emb_kernel(idx_hbm, table_hbm, out_hbm, xblk, rows0, rows1, oblk,
                sem0, sem1):
    wid = lax.axis_index("s") * NC + lax.axis_index("c")

    pltpu.sync_copy(idx_hbm.at[:, pl.ds(wid, 1), :], xblk)

    iota = lax.iota(jnp.int32, 16)
    rowv = [iota + c * 16 for c in range(BW // 16)]

    def fire(h, buf, sem):
        pltpu.async_copy(table_hbm.at[xblk.at[h, 0]], buf, sem)

    def process(h, buf, sem):
        pltpu.make_async_copy(table_hbm.at[pl.ds(0, BW)], buf, sem).wait()
        for c in range(BW // 16):
            rv = rowv[c]
            for d in range(D):
                vals = plsc.load_gather(buf, [rv, jnp.full((16,), d, jnp.int32)])
                oblk[d, pl.ds(c * 16, 16)] = vals
        pltpu.sync_copy(oblk, out_hbm.at[h, :, pl.ds(wid * BW, BW)])

    fire(0, rows0, sem0)
    fire(1, rows1, sem1)

    @pl.loop(0, HIST - 2, step=2)
    def _pair(hh):
        process(hh, rows0, sem0)
        fire(hh + 2, rows0, sem0)
        process(hh + 1, rows1, sem1)
        fire(hh + 3, rows1, sem1)

    process(HIST - 2, rows0, sem0)
    process(HIST - 1, rows1, sem1)


def kernel(x, table):
    idx = x.T.reshape(HIST, NW, BW)
    table_p = jnp.pad(table, ((0, 0), (0, PD - D)))
    out = _emb_kernel(idx, table_p)
    return out.transpose(2, 0, 1)


# speedup vs baseline: 1.7428x; 1.5954x over previous
import functools

import jax
import jax.numpy as jnp
from jax import lax
from jax.experimental import pallas as pl
from jax.experimental.pallas import tpu as pltpu
from jax.experimental.pallas import tpu_sc as plsc

BATCH = 4096
HIST = 200
D = 64
PD = 128

NC, NS = 2, 16
NW = NC * NS
BW = BATCH // NW
VOCAB = 1000000

_mesh = plsc.VectorSubcoreMesh(core_axis_name="c", subcore_axis_name="s")


@functools.partial(
    pl.kernel,
    out_type=jax.ShapeDtypeStruct((HIST, D, BATCH), jnp.float32),
    mesh=_mesh,
    scratch_types=[
        pltpu.VMEM((HIST, 1, BW), jnp.int32),
        pltpu.VMEM((BW, PD), jnp.float32),
        pltpu.VMEM((BW, PD), jnp.float32),
        pltpu.VMEM((D, BW), jnp.float32),
        pltpu.SemaphoreType.DMA,
        pltpu.SemaphoreType.DMA,
    ],
    compiler_params=pltpu.CompilerParams(needs_layout_passes=False),
)
def _emb_kernel(idx_hbm, table_hbm, out_hbm, xblk, rows0, rows1, oblk,
                sem0, sem1):
    wid = lax.axis_index("s") * NC + lax.axis_index("c")

    pltpu.sync_copy(idx_hbm.at[:, pl.ds(wid, 1), :], xblk)

    iota = lax.iota(jnp.int32, 16)
    rowv = [iota + c * 16 for c in range(BW // 16)]

    def fire(h, buf, sem):
        pltpu.async_copy(table_hbm.at[xblk.at[h, 0]], buf, sem)

    def process(h, buf, sem):
        pltpu.make_async_copy(table_hbm.at[pl.ds(0, BW)], buf, sem).wait()
        @plsc.parallel_loop(0, D, unroll=8)
        def _d(d):
            col = jnp.full((16,), 0, jnp.int32) + d
            for c in range(BW // 16):
                oblk[d, pl.ds(c * 16, 16)] = plsc.load_gather(
                    buf, [rowv[c], col])
        pltpu.sync_copy(oblk, out_hbm.at[h, :, pl.ds(wid * BW, BW)])

    fire(0, rows0, sem0)
    fire(1, rows1, sem1)

    @pl.loop(0, HIST - 2, step=2)
    def _pair(hh):
        process(hh, rows0, sem0)
        fire(hh + 2, rows0, sem0)
        process(hh + 1, rows1, sem1)
        fire(hh + 3, rows1, sem1)

    process(HIST - 2, rows0, sem0)
    process(HIST - 1, rows1, sem1)


def kernel(x, table):
    idx = x.T.reshape(HIST, NW, BW)
    table_p = jnp.pad(table, ((0, 0), (0, PD - D)))
    out = _emb_kernel(idx, table_p)
    return out.transpose(2, 0, 1)


# speedup vs baseline: 1.8434x; 1.0577x over previous
import functools

import jax
import jax.numpy as jnp
from jax import lax
from jax.experimental import pallas as pl
from jax.experimental.pallas import tpu as pltpu
from jax.experimental.pallas import tpu_sc as plsc

BATCH = 4096
HIST = 200
D = 64
PD = 128

NC, NS = 2, 16
NW = NC * NS
BW = BATCH // NW
VOCAB = 1000000
NBUF = 4

_mesh = plsc.VectorSubcoreMesh(core_axis_name="c", subcore_axis_name="s")


@functools.partial(
    pl.kernel,
    out_type=jax.ShapeDtypeStruct((HIST, D, BATCH), jnp.float32),
    mesh=_mesh,
    scratch_types=(
        [pltpu.VMEM((HIST, 1, BW), jnp.int32)]
        + [pltpu.VMEM((BW, PD), jnp.float32) for _ in range(NBUF)]
        + [pltpu.VMEM((D, BW), jnp.float32) for _ in range(2)]
        + [pltpu.SemaphoreType.DMA for _ in range(NBUF + 2)]
    ),
    compiler_params=pltpu.CompilerParams(needs_layout_passes=False),
)
def _emb_kernel(idx_hbm, table_hbm, out_hbm, xblk, r0, r1, r2, r3,
                ob0, ob1, sg0, sg1, sg2, sg3, so0, so1):
    wid = lax.axis_index("s") * NC + lax.axis_index("c")
    rows = [r0, r1, r2, r3]
    sgs = [sg0, sg1, sg2, sg3]
    obs = [ob0, ob1]
    sos = [so0, so1]

    pltpu.sync_copy(idx_hbm.at[:, pl.ds(wid, 1), :], xblk)

    iota = lax.iota(jnp.int32, 16)
    rowv = [iota + c * 16 for c in range(BW // 16)]

    def fire(h, k):
        pltpu.async_copy(table_hbm.at[xblk.at[h, 0]], rows[k], sgs[k])

    def wait_gather(k):
        pltpu.make_async_copy(
            table_hbm.at[pl.ds(0, BW)], rows[k], sgs[k]).wait()

    def wait_writeback(p):
        pltpu.make_async_copy(
            obs[p], out_hbm.at[0, :, pl.ds(0, BW)], sos[p]).wait()

    def transpose(k, p):
        buf, ob = rows[k], obs[p]

        @plsc.parallel_loop(0, D, unroll=8)
        def _d(d):
            col = jnp.full((16,), 0, jnp.int32) + d
            for c in range(BW // 16):
                ob[d, pl.ds(c * 16, 16)] = plsc.load_gather(
                    buf, [rowv[c], col])

    def writeback(h, p):
        pltpu.async_copy(obs[p], out_hbm.at[h, :, pl.ds(wid * BW, BW)],
                         sos[p])

    for k in range(NBUF):
        fire(k, k)

    @pl.loop(0, HIST - NBUF, step=NBUF)
    def _quad(hh):
        for k in range(NBUF):
            h = hh + k
            wait_gather(k)

            @pl.when(hh + k >= 2)
            def _():
                wait_writeback(k % 2)

            transpose(k, k % 2)
            writeback(h, k % 2)
            fire(h + NBUF, k)

    for k in range(NBUF):
        h = HIST - NBUF + k
        wait_gather(k)
        wait_writeback(k % 2)
        transpose(k, k % 2)
        writeback(h, k % 2)
    wait_writeback(0)
    wait_writeback(1)


def kernel(x, table):
    idx = x.T.reshape(HIST, NW, BW)
    table_p = jnp.concatenate(
        [table, jnp.zeros((VOCAB, PD - D), table.dtype)], axis=1)
    out = _emb_kernel(idx, table_p)
    return out.transpose(2, 0, 1)
